# baseline (speedup 1.0000x reference)
import functools

import jax
import jax.numpy as jnp
from jax import lax
from jax.experimental import pallas as pl
from jax.experimental.pallas import tpu as pltpu
from jax.experimental.pallas import tpu_sc as plsc

N_FEATURES = 26
NUM_EMBEDDINGS = 100000
EMBEDDING_DIM = 16
BATCH = 16384

NUM_WORKERS = 32
NB = BATCH // NUM_WORKERS
LANES = 16
CHUNKS = (13, 13)
NSLOT = 4

_mesh = plsc.VectorSubcoreMesh(core_axis_name="c", subcore_axis_name="s")


def _make_chunk_kernel(nf, fbase):
    @functools.partial(
        pl.kernel,
        mesh=_mesh,
        out_type=jax.ShapeDtypeStruct((nf, EMBEDDING_DIM, BATCH), jnp.float32),
        scratch_types=[
            pltpu.VMEM((NB, N_FEATURES), jnp.int32),
            pltpu.VMEM((NSLOT, NB), jnp.int32),
            pltpu.VMEM((NSLOT, EMBEDDING_DIM, NB), jnp.float32),
            pltpu.SemaphoreType.DMA,
            pltpu.SemaphoreType.DMA,
        ],
        compiler_params=pltpu.CompilerParams(
            use_tc_tiling_on_sc=False, needs_layout_passes=False
        ),
    )
    def _gather_chunk(x_hbm, wt_hbm, out_hbm, xb_v, idx_v, fbuf_v, gsem, wsem):
        wid = lax.axis_index("s") * 2 + lax.axis_index("c")
        b0 = wid * NB

        pltpu.sync_copy(x_hbm.at[pl.ds(b0, NB), :], xb_v)

        iota16 = jax.lax.iota(jnp.int32, LANES)

        def extract_idx(f, slot):
            colf = jnp.full((LANES,), fbase, jnp.int32) + f

            def body(i, _):
                rows = i * LANES + iota16
                vals = plsc.load_gather(xb_v, [rows, colf])
                idx_v[slot, pl.ds(i * LANES, LANES)] = vals
                return ()

            lax.fori_loop(0, NB // LANES, body, ())

        def fire_gathers(f, slot):
            for d in range(EMBEDDING_DIM):
                row0 = (f * EMBEDDING_DIM + d) * NUM_EMBEDDINGS
                pltpu.async_copy(
                    wt_hbm.at[pl.ds(row0, NUM_EMBEDDINGS)].at[idx_v.at[slot]],
                    fbuf_v.at[slot, d],
                    gsem,
                )

        def drain_gathers(slot):
            pltpu.make_async_copy(
                out_hbm.at[0, :, pl.ds(0, NB)], fbuf_v.at[slot], gsem
            ).wait()

        def fire_write(f, slot):
            pltpu.async_copy(
                fbuf_v.at[slot],
                out_hbm.at[f, :, pl.ds(b0, NB)],
                wsem,
            )

        def drain_write(slot):
            pltpu.make_async_copy(
                fbuf_v.at[slot], out_hbm.at[0, :, pl.ds(0, NB)], wsem
            ).wait()

        extract_idx(0, 0)
        extract_idx(1, 1)
        fire_gathers(0, 0)

        def step(f, _):
            slot = lax.rem(f, NSLOT)

            @pl.when(f >= 3)
            def _():
                drain_write(lax.rem(f + 1, NSLOT))

            @pl.when(f + 1 < nf)
            def _():
                fire_gathers(f + 1, lax.rem(f + 1, NSLOT))

            @pl.when(f + 2 < nf)
            def _():
                extract_idx(f + 2, lax.rem(f + 2, NSLOT))

            drain_gathers(slot)
            fire_write(f, slot)
            return ()

        lax.fori_loop(0, nf, step, ())
        for t in range(min(3, nf)):
            drain_write((nf - min(3, nf) + t) % NSLOT)

    return _gather_chunk


_chunk_kernels = []
_base = 0
for _nf in CHUNKS:
    _chunk_kernels.append((_make_chunk_kernel(_nf, _base), _base, _nf))
    _base += _nf


def kernel(x, embedding_weights):
    wt = jnp.transpose(embedding_weights, (0, 2, 1))
    outs = []
    for fn, fbase, nf in _chunk_kernels:
        wflat = lax.slice_in_dim(wt, fbase, fbase + nf, axis=0).reshape(-1)
        outs.append(fn(x, wflat))
    out_t = jnp.concatenate(outs, axis=0)
    return jnp.transpose(out_t, (2, 0, 1))


# speedup vs baseline: 1.0835x; 1.0835x over previous
import functools

import jax
import jax.numpy as jnp
from jax import lax
from jax.experimental import pallas as pl
from jax.experimental.pallas import tpu as pltpu
from jax.experimental.pallas import tpu_sc as plsc

N_FEATURES = 26
NUM_EMBEDDINGS = 100000
EMBEDDING_DIM = 16
BATCH = 16384

NUM_WORKERS = 32
NB = BATCH // NUM_WORKERS
LANES = 16
CHUNKS = (7, 8, 8, 3)
NSLOT = 4

_mesh = plsc.VectorSubcoreMesh(core_axis_name="c", subcore_axis_name="s")


def _make_chunk_kernel(nf, fbase):
    @functools.partial(
        pl.kernel,
        mesh=_mesh,
        out_type=jax.ShapeDtypeStruct((nf, EMBEDDING_DIM, BATCH), jnp.float32),
        scratch_types=[
            pltpu.VMEM((NB, N_FEATURES), jnp.int32),
            pltpu.VMEM((NSLOT, NB), jnp.int32),
            pltpu.VMEM((NSLOT, EMBEDDING_DIM, NB), jnp.float32),
            pltpu.SemaphoreType.DMA,
            pltpu.SemaphoreType.DMA,
        ],
        compiler_params=pltpu.CompilerParams(
            use_tc_tiling_on_sc=False, needs_layout_passes=False
        ),
    )
    def _gather_chunk(x_hbm, wt_hbm, out_hbm, xb_v, idx_v, fbuf_v, gsem, wsem):
        wid = lax.axis_index("s") * 2 + lax.axis_index("c")
        b0 = wid * NB

        pltpu.sync_copy(x_hbm.at[pl.ds(b0, NB), :], xb_v)

        iota16 = jax.lax.iota(jnp.int32, LANES)

        def extract_idx(f, slot):
            colf = jnp.full((LANES,), fbase, jnp.int32) + f

            def body(i, _):
                rows = i * LANES + iota16
                vals = plsc.load_gather(xb_v, [rows, colf])
                idx_v[slot, pl.ds(i * LANES, LANES)] = vals
                return ()

            lax.fori_loop(0, NB // LANES, body, ())

        def fire_gathers(f, slot):
            for d in range(EMBEDDING_DIM):
                row0 = (f * EMBEDDING_DIM + d) * NUM_EMBEDDINGS
                pltpu.async_copy(
                    wt_hbm.at[pl.ds(row0, NUM_EMBEDDINGS)].at[idx_v.at[slot]],
                    fbuf_v.at[slot, d],
                    gsem,
                )

        def drain_gathers(slot):
            pltpu.make_async_copy(
                out_hbm.at[0, :, pl.ds(0, NB)], fbuf_v.at[slot], gsem
            ).wait()

        def fire_write(f, slot):
            pltpu.async_copy(
                fbuf_v.at[slot],
                out_hbm.at[f, :, pl.ds(b0, NB)],
                wsem,
            )

        def drain_write(slot):
            pltpu.make_async_copy(
                fbuf_v.at[slot], out_hbm.at[0, :, pl.ds(0, NB)], wsem
            ).wait()

        extract_idx(0, 0)
        extract_idx(1, 1)
        fire_gathers(0, 0)

        def step(f, _):
            slot = lax.rem(f, NSLOT)

            @pl.when(f >= 3)
            def _():
                drain_write(lax.rem(f + 1, NSLOT))

            @pl.when(f + 1 < nf)
            def _():
                fire_gathers(f + 1, lax.rem(f + 1, NSLOT))

            @pl.when(f + 2 < nf)
            def _():
                extract_idx(f + 2, lax.rem(f + 2, NSLOT))

            drain_gathers(slot)
            fire_write(f, slot)
            return ()

        lax.fori_loop(0, nf, step, ())
        for t in range(min(3, nf)):
            drain_write((nf - min(3, nf) + t) % NSLOT)

    return _gather_chunk


_chunk_kernels = []
_base = 0
for _nf in CHUNKS:
    _chunk_kernels.append((_make_chunk_kernel(_nf, _base), _base, _nf))
    _base += _nf


def kernel(x, embedding_weights):
    wt = jnp.transpose(embedding_weights, (0, 2, 1))
    outs = []
    for fn, fbase, nf in _chunk_kernels:
        wflat = lax.slice_in_dim(wt, fbase, fbase + nf, axis=0).reshape(-1)
        outs.append(fn(x, wflat))
    out_t = jnp.concatenate(outs, axis=0)
    return jnp.transpose(out_t, (2, 0, 1))


# speedup vs baseline: 1.0876x; 1.0038x over previous
import functools

import jax
import jax.numpy as jnp
from jax import lax
from jax.experimental import pallas as pl
from jax.experimental.pallas import tpu as pltpu
from jax.experimental.pallas import tpu_sc as plsc

N_FEATURES = 26
NUM_EMBEDDINGS = 100000
EMBEDDING_DIM = 16
BATCH = 16384

NUM_WORKERS = 32
NB = BATCH // NUM_WORKERS
LANES = 16
CHUNKS = (5, 6, 6, 6, 3)
NSLOT = 4

_mesh = plsc.VectorSubcoreMesh(core_axis_name="c", subcore_axis_name="s")


def _make_chunk_kernel(nf, fbase):
    @functools.partial(
        pl.kernel,
        mesh=_mesh,
        out_type=jax.ShapeDtypeStruct((nf, EMBEDDING_DIM, BATCH), jnp.float32),
        scratch_types=[
            pltpu.VMEM((NB, N_FEATURES), jnp.int32),
            pltpu.VMEM((NSLOT, NB), jnp.int32),
            pltpu.VMEM((NSLOT, EMBEDDING_DIM, NB), jnp.float32),
            pltpu.SemaphoreType.DMA,
            pltpu.SemaphoreType.DMA,
        ],
        compiler_params=pltpu.CompilerParams(
            use_tc_tiling_on_sc=False, needs_layout_passes=False
        ),
    )
    def _gather_chunk(x_hbm, wt_hbm, out_hbm, xb_v, idx_v, fbuf_v, gsem, wsem):
        wid = lax.axis_index("s") * 2 + lax.axis_index("c")
        b0 = wid * NB

        pltpu.sync_copy(x_hbm.at[pl.ds(b0, NB), :], xb_v)

        iota16 = jax.lax.iota(jnp.int32, LANES)

        def extract_idx(f, slot):
            colf = jnp.full((LANES,), fbase, jnp.int32) + f

            def body(i, _):
                rows = i * LANES + iota16
                vals = plsc.load_gather(xb_v, [rows, colf])
                idx_v[slot, pl.ds(i * LANES, LANES)] = vals
                return ()

            lax.fori_loop(0, NB // LANES, body, ())

        def fire_gathers(f, slot):
            for d in range(EMBEDDING_DIM):
                row0 = (f * EMBEDDING_DIM + d) * NUM_EMBEDDINGS
                pltpu.async_copy(
                    wt_hbm.at[pl.ds(row0, NUM_EMBEDDINGS)].at[idx_v.at[slot]],
                    fbuf_v.at[slot, d],
                    gsem,
                )

        def drain_gathers(slot):
            pltpu.make_async_copy(
                out_hbm.at[0, :, pl.ds(0, NB)], fbuf_v.at[slot], gsem
            ).wait()

        def fire_write(f, slot):
            pltpu.async_copy(
                fbuf_v.at[slot],
                out_hbm.at[f, :, pl.ds(b0, NB)],
                wsem,
            )

        def drain_write(slot):
            pltpu.make_async_copy(
                fbuf_v.at[slot], out_hbm.at[0, :, pl.ds(0, NB)], wsem
            ).wait()

        extract_idx(0, 0)
        extract_idx(1, 1)
        fire_gathers(0, 0)

        def step(f, _):
            slot = lax.rem(f, NSLOT)

            @pl.when(f >= 3)
            def _():
                drain_write(lax.rem(f + 1, NSLOT))

            @pl.when(f + 1 < nf)
            def _():
                fire_gathers(f + 1, lax.rem(f + 1, NSLOT))

            @pl.when(f + 2 < nf)
            def _():
                extract_idx(f + 2, lax.rem(f + 2, NSLOT))

            drain_gathers(slot)
            fire_write(f, slot)
            return ()

        lax.fori_loop(0, nf, step, ())
        for t in range(min(3, nf)):
            drain_write((nf - min(3, nf) + t) % NSLOT)

    return _gather_chunk


_chunk_kernels = []
_base = 0
for _nf in CHUNKS:
    _chunk_kernels.append((_make_chunk_kernel(_nf, _base), _base, _nf))
    _base += _nf


def kernel(x, embedding_weights):
    wt = jnp.transpose(embedding_weights, (0, 2, 1))
    outs = []
    for fn, fbase, nf in _chunk_kernels:
        wflat = lax.slice_in_dim(wt, fbase, fbase + nf, axis=0).reshape(-1)
        outs.append(fn(x, wflat))
    out_t = jnp.concatenate(outs, axis=0)
    return jnp.transpose(out_t, (2, 0, 1))
